# baseline (speedup 1.0000x reference)
import functools

import jax
import jax.numpy as jnp
from jax import lax
from jax.experimental import pallas as pl
from jax.experimental.pallas import tpu as pltpu
from jax.experimental.pallas import tpu_sc as plsc

_NC = 2
_NS = 16
_NW = _NC * _NS
_U = 128


def _sc_gather_sum(ctab, xtab, cidx, xidx, nidx, B, BN, D):
    uc = B // _U // _NW
    un = BN // _U // _NW
    nd16 = D // 16

    mesh = plsc.VectorSubcoreMesh(core_axis_name="c", subcore_axis_name="s")

    @functools.partial(
        pl.kernel,
        mesh=mesh,
        out_type=[
            jax.ShapeDtypeStruct((B, D), jnp.float32),
            jax.ShapeDtypeStruct((B, D), jnp.float32),
            jax.ShapeDtypeStruct((BN, D), jnp.float32),
        ],
        scratch_types=[
            pltpu.VMEM((_U,), jnp.int32),
            pltpu.VMEM((_U,), jnp.int32),
            pltpu.VMEM((_U, D), jnp.float32),
            pltpu.VMEM((_U, D), jnp.float32),
            pltpu.SemaphoreType.DMA,
        ],
        compiler_params=pltpu.CompilerParams(use_tc_tiling_on_sc=False),
    )
    def sc_fn(ctab_h, xtab_h, cidx_h, xidx_h, nidx_h,
              csum_h, xsum_h, nsum_h,
              idx0, idx1, bufa, bufb, sem):
        wid = lax.axis_index("s") * _NC + lax.axis_index("c")

        def run_units(tab_h, idx_h, out_h, base_u, n_units):
            def unit(j, carry):
                u = base_u + j
                pltpu.sync_copy(idx_h.at[0, u], idx0)
                pltpu.sync_copy(idx_h.at[1, u], idx1)
                pltpu.async_copy(tab_h.at[idx0], bufa, sem).wait()
                pltpu.async_copy(tab_h.at[idx1], bufb, sem).wait()

                def sum_row(r, c2):
                    for c in range(nd16):
                        sl = pl.ds(c * 16, 16)
                        bufa[r, sl] = bufa[r, sl] + bufb[r, sl]
                    return c2

                lax.fori_loop(0, _U, sum_row, 0)
                pltpu.sync_copy(bufa, out_h.at[pl.ds(u * _U, _U)])
                return carry

            lax.fori_loop(0, n_units, unit, 0)

        run_units(ctab_h, cidx_h, csum_h, wid * uc, uc)
        run_units(xtab_h, xidx_h, xsum_h, wid * uc, uc)
        run_units(xtab_h, nidx_h, nsum_h, wid * un, un)

    return sc_fn(ctab, xtab, cidx, xidx, nidx)


def _tc_loss(csum, xsum, nsum, B, BN, D, neg):
    grid = 32
    r = B // grid

    def tc_fn(c_ref, x_ref, n_ref, o_ref):
        i = pl.program_id(0)
        c = c_ref[...]
        x = x_ref[...]
        ng = n_ref[...].reshape(r, neg, D)
        score = 0.25 * jnp.sum(c * x, axis=1)
        score = jnp.clip(score, -10.0, 10.0)
        pos_loss = jnp.log(1.0 + jnp.exp(-score))
        nd = 0.25 * jnp.sum(ng * c[:, None, :], axis=2)
        nd = jnp.clip(nd, -10.0, 10.0)
        neg_loss = jnp.sum(jnp.log(1.0 + jnp.exp(nd)), axis=1)
        part = jnp.sum(pos_loss + neg_loss)

        @pl.when(i == 0)
        def _():
            o_ref[0, 0] = 0.0

        o_ref[0, 0] += part

        @pl.when(i == grid - 1)
        def _():
            o_ref[0, 0] = o_ref[0, 0] * (1.0 / B)

    out = pl.pallas_call(
        tc_fn,
        grid=(grid,),
        in_specs=[
            pl.BlockSpec((r, D), lambda i: (i, 0)),
            pl.BlockSpec((r, D), lambda i: (i, 0)),
            pl.BlockSpec((r * neg, D), lambda i: (i, 0)),
        ],
        out_specs=pl.BlockSpec((1, 1), lambda i: (0, 0),
                               memory_space=pltpu.SMEM),
        out_shape=jax.ShapeDtypeStruct((1, 1), jnp.float32),
        compiler_params=pltpu.CompilerParams(
            dimension_semantics=("arbitrary",)),
    )(csum, xsum, nsum)
    return out[0, 0]


def kernel(centers, contexts, neg_contexts, center_emb, context_emb):
    F, V, D = center_emb.shape
    B = centers.shape[0]
    BN = neg_contexts.shape[0]
    neg = BN // B

    ctab = center_emb.reshape(F * V, D)
    xtab = context_emb.reshape(F * V, D)
    offs = jnp.arange(F, dtype=jnp.int32) * V
    cidx = (centers + offs[None, :]).T.reshape(F, B // _U, _U)
    xidx = (contexts + offs[None, :]).T.reshape(F, B // _U, _U)
    nidx = (neg_contexts + offs[None, :]).T.reshape(F, BN // _U, _U)

    csum, xsum, nsum = _sc_gather_sum(ctab, xtab, cidx, xidx, nidx, B, BN, D)
    return _tc_loss(csum, xsum, nsum, B, BN, D, neg)
